# baseline (speedup 1.0000x reference)
import functools

import jax
import jax.numpy as jnp
from jax import lax
from jax.experimental import pallas as pl
from jax.experimental.pallas import tpu as pltpu
from jax.experimental.pallas import tpu_sc as plsc

B, C, H, W = 32, 32, 224, 224

_mesh = plsc.VectorSubcoreMesh(core_axis_name="c", subcore_axis_name="s")


@functools.partial(
    pl.kernel,
    out_type=jax.ShapeDtypeStruct((B * C, H, W), jnp.float32),
    mesh=_mesh,
    scratch_types=[
        pltpu.VMEM((B + 16,), jnp.int32),
        pltpu.VMEM((1, H, W), jnp.float32),
        pltpu.SemaphoreType.DMA,
        pltpu.SemaphoreType.DMA,
    ],
)
def _sc_gather_bcast(tab_hbm, idx_hbm, out_hbm, idx_v, plane_v, gsem, wsem):
    wid = lax.axis_index("s") * 2 + lax.axis_index("c")
    pltpu.sync_copy(idx_hbm, idx_v.at[pl.ds(0, B)])
    src = wid * C + idx_v[pl.ds(wid, 16)][0]
    pltpu.async_copy(tab_hbm.at[pl.ds(src, 1)], plane_v, gsem).wait()
    NDIRECT = 4
    copies = [
        pltpu.async_copy(plane_v, out_hbm.at[pl.ds(wid * C + c, 1)], wsem)
        for c in range(C - NDIRECT)
    ]
    copies += [
        pltpu.async_copy(
            tab_hbm.at[pl.ds(src, 1)],
            out_hbm.at[pl.ds(wid * C + c, 1)], wsem)
        for c in range(C - NDIRECT, C)
    ]
    for cp in copies:
        cp.wait()


def kernel(logits, indices):
    tab = logits.reshape(B * C, H, W)
    idx = indices.astype(jnp.int32)
    out = _sc_gather_bcast(tab, idx)
    return out.reshape(B, C, H, W)
